# baseline (speedup 1.0000x reference)
import functools

import jax
import jax.numpy as jnp
from jax.experimental import pallas as pl
from jax.experimental.pallas import tpu as pltpu

_H = 4
_LANE = 128
_BITS_ONE_PLUS = 0x3F800001


def _exclusive_prefix(tie_f, rows, cols):
    chunks = cols // _LANE
    t3 = tie_f.reshape(rows * chunks, _LANE)
    li = jax.lax.broadcasted_iota(jnp.int32, (_LANE, _LANE), 0)
    lj = jax.lax.broadcasted_iota(jnp.int32, (_LANE, _LANE), 1)
    m = (li < lj).astype(jnp.float32)
    within = jnp.dot(t3, m, preferred_element_type=jnp.float32)
    within = within.reshape(rows, chunks, _LANE)
    csum = jnp.sum(t3.reshape(rows, chunks, _LANE), axis=2)
    ci = jax.lax.broadcasted_iota(jnp.int32, (chunks, chunks), 0)
    cj = jax.lax.broadcasted_iota(jnp.int32, (chunks, chunks), 1)
    mc = (ci < cj).astype(jnp.float32)
    excl = jnp.dot(csum, mc, preferred_element_type=jnp.float32)
    prefix = within + excl[:, :, None]
    return prefix.reshape(rows, cols)


def _block_body(fw_ref, invt_ref, e1_ref, e2_ref, out_ref, *, topk):
    rows, cols = out_ref.shape
    srows = _H * rows
    kf = jnp.float32(topk)

    es = []
    zs = []
    for h in range(_H):
        x = jnp.dot(e1_ref[h], e2_ref[h], preferred_element_type=jnp.float32)
        x = jnp.maximum(x, 0.0) * invt_ref[h]
        m = jnp.max(x, axis=1, keepdims=True)
        e = jnp.exp(x - m)
        es.append(e)
        zs.append(jnp.sum(e, axis=1, keepdims=True))
    e_all = jnp.concatenate(es, axis=0)
    z_all = jnp.concatenate(zs, axis=0)
    fw_all = jnp.concatenate(
        [jnp.full((rows, 1), fw_ref[h], jnp.float32) for h in range(_H)], axis=0
    )

    chunks = cols // _LANE
    if chunks >= topk:
        cmax = jnp.max(e_all.reshape(srows, chunks, _LANE), axis=2)
        lo0 = jax.lax.bitcast_convert_type(
            jnp.min(cmax, axis=1, keepdims=True), jnp.int32
        )
    else:
        lo0 = jnp.zeros((srows, 1), jnp.int32)
    hi0 = jnp.full((srows, 1), _BITS_ONE_PLUS, jnp.int32)

    def _cond(carry):
        return carry[-1] > 0

    def _step1(carry):
        lo, hi, _ = carry
        mid = lo + jax.lax.shift_right_logical(hi - lo, 1)
        mid_f = jax.lax.bitcast_convert_type(mid, jnp.float32)
        cnt = jnp.sum((e_all >= mid_f).astype(jnp.float32), axis=1, keepdims=True)
        ok = cnt >= kf
        lo = jnp.where(ok, mid, lo)
        hi = jnp.where(ok, hi, mid)
        cont = (jnp.max(hi - lo).astype(jnp.int32) >= 32768).astype(jnp.int32)
        return lo, hi, cont

    lo1, hi1, _ = jax.lax.while_loop(_cond, _step1, (lo0, hi0, jnp.int32(1)))

    eb = jax.lax.bitcast_convert_type(e_all, jnp.int32)
    r16 = jnp.clip(eb - lo1, 0, 32767).astype(jnp.int16)

    def _step2(carry):
        lo, hi, _ = carry
        mid = lo + jax.lax.shift_right_logical(hi - lo, 1)
        mid16 = jnp.clip(mid - lo1, 0, 32767).astype(jnp.int16)
        mask16 = jnp.where(r16 >= mid16, jnp.int16(1), jnp.int16(0))
        cnt = jnp.sum(mask16, axis=1, keepdims=True, dtype=jnp.int32)
        ok = cnt >= topk
        lo = jnp.where(ok, mid, lo)
        hi = jnp.where(ok, hi, mid)
        cont = (jnp.max(hi - lo).astype(jnp.int32) > 1).astype(jnp.int32)
        return lo, hi, cont

    tb, _, _ = jax.lax.while_loop(_cond, _step2, (lo1, hi1, jnp.int32(1)))
    tb_f = jax.lax.bitcast_convert_type(tb, jnp.float32)

    gt = e_all > tb_f
    tie = e_all == tb_f
    c_gt = jnp.sum(gt.astype(jnp.float32), axis=1, keepdims=True)
    k_rem = kf - c_gt
    prefix = _exclusive_prefix(tie.astype(jnp.float32), srows, cols)
    sel_mask = gt | (tie & (prefix < k_rem))

    sel = jnp.where(sel_mask, e_all, 0.0)
    s = jnp.sum(sel, axis=1, keepdims=True)
    scale = fw_all / (s + 1e-8 * z_all)
    weighted = sel * scale
    w4 = weighted.reshape(_H, rows, cols)
    out_ref[...] = w4[0] + w4[1] + w4[2] + w4[3]


def _fused_topk_adj(e1, e2, fw, invt, *, topk, block_rows):
    h, n, d = e1.shape
    grid = (n // block_rows,)
    return pl.pallas_call(
        functools.partial(_block_body, topk=topk),
        grid=grid,
        in_specs=[
            pl.BlockSpec(memory_space=pltpu.SMEM),
            pl.BlockSpec(memory_space=pltpu.SMEM),
            pl.BlockSpec((h, block_rows, d), lambda i: (0, i, 0)),
            pl.BlockSpec((h, d, n), lambda i: (0, 0, 0)),
        ],
        out_specs=pl.BlockSpec((block_rows, n), lambda i: (i, 0)),
        out_shape=jax.ShapeDtypeStruct((n, n), jnp.float32),
    )(fw, invt, e1, e2)


def kernel(node_embeddings1, node_embeddings2, temperature, fusion_weights):
    temp = jnp.clip(temperature, 0.1, 2.0)
    invt = 1.0 / temp
    fw = jax.nn.softmax(fusion_weights, axis=0)
    return _fused_topk_adj(
        node_embeddings1,
        node_embeddings2,
        fw,
        invt,
        topk=32,
        block_rows=128,
    )


# speedup vs baseline: 1.2346x; 1.2346x over previous
import functools

import jax
import jax.numpy as jnp
from jax.experimental import pallas as pl
from jax.experimental.pallas import tpu as pltpu

_H = 4
_LANE = 128
_BITS_ONE_PLUS = 0x3F800001


def _exclusive_prefix(tie_f, rows, cols):
    chunks = cols // _LANE
    t3 = tie_f.reshape(rows * chunks, _LANE)
    li = jax.lax.broadcasted_iota(jnp.int32, (_LANE, _LANE), 0)
    lj = jax.lax.broadcasted_iota(jnp.int32, (_LANE, _LANE), 1)
    m = (li < lj).astype(jnp.float32)
    within = jnp.dot(t3, m, preferred_element_type=jnp.float32)
    within = within.reshape(rows, chunks, _LANE)
    csum = jnp.sum(t3.reshape(rows, chunks, _LANE), axis=2)
    ci = jax.lax.broadcasted_iota(jnp.int32, (chunks, chunks), 0)
    cj = jax.lax.broadcasted_iota(jnp.int32, (chunks, chunks), 1)
    mc = (ci < cj).astype(jnp.float32)
    excl = jnp.dot(csum, mc, preferred_element_type=jnp.float32)
    prefix = within + excl[:, :, None]
    return prefix.reshape(rows, cols)


def _block_body(fw_ref, invt_ref, e1_ref, e2_ref, out_ref, *, topk):
    rows, cols = out_ref.shape
    srows = _H * rows
    kf = jnp.float32(topk)

    es = []
    zs = []
    for h in range(_H):
        x = jnp.dot(e1_ref[h], e2_ref[h], preferred_element_type=jnp.float32)
        x = jnp.maximum(x, 0.0) * invt_ref[h]
        m = jnp.max(x, axis=1, keepdims=True)
        e = jnp.exp(x - m)
        es.append(e)
        zs.append(jnp.sum(e, axis=1, keepdims=True))
    e_all = jnp.concatenate(es, axis=0)
    z_all = jnp.concatenate(zs, axis=0)
    fw_all = jnp.concatenate(
        [jnp.full((rows, 1), fw_ref[h], jnp.float32) for h in range(_H)], axis=0
    )

    chunks = cols // _LANE
    if chunks >= topk:
        cmax = jnp.max(e_all.reshape(srows, chunks, _LANE), axis=2)
        lo0 = jax.lax.bitcast_convert_type(
            jnp.min(cmax, axis=1, keepdims=True), jnp.int32
        )
    else:
        lo0 = jnp.zeros((srows, 1), jnp.int32)
    hi0 = jnp.full((srows, 1), _BITS_ONE_PLUS, jnp.int32)

    def _cond(carry):
        return carry[-1] > 0

    def _step(carry):
        lo, hi, _ = carry
        mid = lo + jax.lax.shift_right_logical(hi - lo, 1)
        mid_f = jax.lax.bitcast_convert_type(mid, jnp.float32)
        cnt = jnp.sum((e_all >= mid_f).astype(jnp.float32), axis=1, keepdims=True)
        ok = cnt >= kf
        lo = jnp.where(ok, mid, lo)
        hi = jnp.where(ok, hi, mid)
        cont = (jnp.max(hi - lo).astype(jnp.int32) > 1).astype(jnp.int32)
        return lo, hi, cont

    tb, _, _ = jax.lax.while_loop(_cond, _step, (lo0, hi0, jnp.int32(1)))
    tb_f = jax.lax.bitcast_convert_type(tb, jnp.float32)

    gt = e_all > tb_f
    tie = e_all == tb_f
    c_gt = jnp.sum(gt.astype(jnp.float32), axis=1, keepdims=True)
    k_rem = kf - c_gt
    prefix = _exclusive_prefix(tie.astype(jnp.float32), srows, cols)
    sel_mask = gt | (tie & (prefix < k_rem))

    sel = jnp.where(sel_mask, e_all, 0.0)
    s = jnp.sum(sel, axis=1, keepdims=True)
    scale = fw_all / (s + 1e-8 * z_all)
    weighted = sel * scale
    w4 = weighted.reshape(_H, rows, cols)
    out_ref[...] = w4[0] + w4[1] + w4[2] + w4[3]


def _fused_topk_adj(e1, e2, fw, invt, *, topk, block_rows):
    h, n, d = e1.shape
    grid = (n // block_rows,)
    return pl.pallas_call(
        functools.partial(_block_body, topk=topk),
        grid=grid,
        in_specs=[
            pl.BlockSpec(memory_space=pltpu.SMEM),
            pl.BlockSpec(memory_space=pltpu.SMEM),
            pl.BlockSpec((h, block_rows, d), lambda i: (0, i, 0)),
            pl.BlockSpec((h, d, n), lambda i: (0, 0, 0)),
        ],
        out_specs=pl.BlockSpec((block_rows, n), lambda i: (i, 0)),
        out_shape=jax.ShapeDtypeStruct((n, n), jnp.float32),
    )(fw, invt, e1, e2)


def kernel(node_embeddings1, node_embeddings2, temperature, fusion_weights):
    temp = jnp.clip(temperature, 0.1, 2.0)
    invt = 1.0 / temp
    fw = jax.nn.softmax(fusion_weights, axis=0)
    return _fused_topk_adj(
        node_embeddings1,
        node_embeddings2,
        fw,
        invt,
        topk=32,
        block_rows=128,
    )


# speedup vs baseline: 1.2360x; 1.0011x over previous
import functools

import jax
import jax.numpy as jnp
from jax.experimental import pallas as pl
from jax.experimental.pallas import tpu as pltpu

_H = 4
_LANE = 128
_BITS_ONE_PLUS = 0x3F800001


def _exclusive_prefix(tie_f, rows, cols):
    chunks = cols // _LANE
    t3 = tie_f.reshape(rows * chunks, _LANE)
    li = jax.lax.broadcasted_iota(jnp.int32, (_LANE, _LANE), 0)
    lj = jax.lax.broadcasted_iota(jnp.int32, (_LANE, _LANE), 1)
    m = (li < lj).astype(jnp.float32)
    within = jnp.dot(t3, m, preferred_element_type=jnp.float32)
    within = within.reshape(rows, chunks, _LANE)
    csum = jnp.sum(t3.reshape(rows, chunks, _LANE), axis=2)
    ci = jax.lax.broadcasted_iota(jnp.int32, (chunks, chunks), 0)
    cj = jax.lax.broadcasted_iota(jnp.int32, (chunks, chunks), 1)
    mc = (ci < cj).astype(jnp.float32)
    excl = jnp.dot(csum, mc, preferred_element_type=jnp.float32)
    prefix = within + excl[:, :, None]
    return prefix.reshape(rows, cols)


def _block_body(fw_ref, invt_ref, e1_ref, e2_ref, out_ref, e_scr, *, topk):
    rows, cols = out_ref.shape
    srows = _H * rows
    kf = jnp.float32(topk)

    zs = []
    for h in range(_H):
        x = jnp.dot(e1_ref[h], e2_ref[h], preferred_element_type=jnp.float32)
        x = jnp.maximum(x, 0.0) * invt_ref[h]
        m = jnp.max(x, axis=1, keepdims=True)
        e = jnp.exp(x - m)
        e_scr[h * rows : (h + 1) * rows, :] = e
        zs.append(jnp.sum(e, axis=1, keepdims=True))
    e_all = e_scr[...]
    z_all = jnp.concatenate(zs, axis=0)
    fw_all = jnp.concatenate(
        [jnp.full((rows, 1), fw_ref[h], jnp.float32) for h in range(_H)], axis=0
    )

    chunks = cols // _LANE
    if chunks >= topk:
        cmax = jnp.max(e_all.reshape(srows, chunks, _LANE), axis=2)
        lo0 = jax.lax.bitcast_convert_type(
            jnp.min(cmax, axis=1, keepdims=True), jnp.int32
        )
    else:
        lo0 = jnp.zeros((srows, 1), jnp.int32)
    hi0 = jnp.full((srows, 1), _BITS_ONE_PLUS, jnp.int32)

    def _cond(carry):
        return carry[-1] > 0

    def _step(carry):
        lo, hi, _ = carry
        mid = lo + jax.lax.shift_right_logical(hi - lo, 1)
        mid_f = jax.lax.bitcast_convert_type(mid, jnp.float32)
        cnt = jnp.sum((e_all >= mid_f).astype(jnp.float32), axis=1, keepdims=True)
        ok = cnt >= kf
        lo = jnp.where(ok, mid, lo)
        hi = jnp.where(ok, hi, mid)
        cont = (jnp.max(hi - lo).astype(jnp.int32) > 1).astype(jnp.int32)
        return lo, hi, cont

    tb, _, _ = jax.lax.while_loop(_cond, _step, (lo0, hi0, jnp.int32(1)))
    tb_f = jax.lax.bitcast_convert_type(tb, jnp.float32)

    gt = e_all > tb_f
    tie = e_all == tb_f
    c_gt = jnp.sum(gt.astype(jnp.float32), axis=1, keepdims=True)
    k_rem = kf - c_gt
    prefix = _exclusive_prefix(tie.astype(jnp.float32), srows, cols)
    sel_mask = gt | (tie & (prefix < k_rem))

    sel = jnp.where(sel_mask, e_all, 0.0)
    s = jnp.sum(sel, axis=1, keepdims=True)
    scale = fw_all / (s + 1e-8 * z_all)
    weighted = sel * scale
    w4 = weighted.reshape(_H, rows, cols)
    out_ref[...] = w4[0] + w4[1] + w4[2] + w4[3]


def _fused_topk_adj(e1, e2, fw, invt, *, topk, block_rows):
    h, n, d = e1.shape
    grid = (n // block_rows,)
    return pl.pallas_call(
        functools.partial(_block_body, topk=topk),
        grid=grid,
        in_specs=[
            pl.BlockSpec(memory_space=pltpu.SMEM),
            pl.BlockSpec(memory_space=pltpu.SMEM),
            pl.BlockSpec((h, block_rows, d), lambda i: (0, i, 0)),
            pl.BlockSpec((h, d, n), lambda i: (0, 0, 0)),
        ],
        out_specs=pl.BlockSpec((block_rows, n), lambda i: (i, 0)),
        out_shape=jax.ShapeDtypeStruct((n, n), jnp.float32),
        scratch_shapes=[pltpu.VMEM((h * block_rows, n), jnp.float32)],
    )(fw, invt, e1, e2)


def kernel(node_embeddings1, node_embeddings2, temperature, fusion_weights):
    temp = jnp.clip(temperature, 0.1, 2.0)
    invt = 1.0 / temp
    fw = jax.nn.softmax(fusion_weights, axis=0)
    return _fused_topk_adj(
        node_embeddings1,
        node_embeddings2,
        fw,
        invt,
        topk=32,
        block_rows=128,
    )


# speedup vs baseline: 1.2960x; 1.0485x over previous
import functools

import jax
import jax.numpy as jnp
from jax.experimental import pallas as pl
from jax.experimental.pallas import tpu as pltpu

_H = 4
_LANE = 128
_BITS_ONE_PLUS = 0x3F800001


def _exclusive_prefix(tie_f, rows, cols):
    chunks = cols // _LANE
    t3 = tie_f.reshape(rows * chunks, _LANE)
    li = jax.lax.broadcasted_iota(jnp.int32, (_LANE, _LANE), 0)
    lj = jax.lax.broadcasted_iota(jnp.int32, (_LANE, _LANE), 1)
    m = (li < lj).astype(jnp.bfloat16)
    within = jnp.dot(t3, m, preferred_element_type=jnp.float32)
    within = within.reshape(rows, chunks, _LANE)
    csum = jnp.sum(
        t3.reshape(rows, chunks, _LANE), axis=2, dtype=jnp.float32
    )
    ci = jax.lax.broadcasted_iota(jnp.int32, (chunks, chunks), 0)
    cj = jax.lax.broadcasted_iota(jnp.int32, (chunks, chunks), 1)
    mc = (ci < cj).astype(jnp.float32)
    excl = jnp.dot(csum, mc, preferred_element_type=jnp.float32)
    prefix = within + excl[:, :, None]
    return prefix.reshape(rows, cols)


def _block_body(fw_ref, invt_ref, e1_ref, e2_ref, out_ref, e_scr, *, topk):
    rows, cols = out_ref.shape
    srows = _H * rows
    kf = jnp.float32(topk)

    zs = []
    for h in range(_H):
        x = jnp.dot(e1_ref[h], e2_ref[h], preferred_element_type=jnp.float32)
        x = jnp.maximum(x, 0.0) * invt_ref[h]
        m = jnp.max(x, axis=1, keepdims=True)
        e = jnp.exp(x - m)
        e_scr[h * rows : (h + 1) * rows, :] = e
        zs.append(jnp.sum(e, axis=1, keepdims=True))
    e_all = e_scr[...]
    z_all = jnp.concatenate(zs, axis=0)
    fw_all = jnp.concatenate(
        [jnp.full((rows, 1), fw_ref[h], jnp.float32) for h in range(_H)], axis=0
    )

    chunks = cols // _LANE
    if chunks >= topk:
        cmax = jnp.max(e_all.reshape(srows, chunks, _LANE), axis=2)
        lo0 = jax.lax.bitcast_convert_type(
            jnp.min(cmax, axis=1, keepdims=True), jnp.int32
        )
    else:
        lo0 = jnp.zeros((srows, 1), jnp.int32)
    hi0 = jnp.full((srows, 1), _BITS_ONE_PLUS, jnp.int32)

    def _cond(carry):
        return carry[-1] > 0

    def _step(carry):
        lo, hi, _ = carry
        mid = lo + jax.lax.shift_right_logical(hi - lo, 1)
        mid_f = jax.lax.bitcast_convert_type(mid, jnp.float32)
        cnt = jnp.sum((e_all >= mid_f).astype(jnp.float32), axis=1, keepdims=True)
        ok = cnt >= kf
        lo = jnp.where(ok, mid, lo)
        hi = jnp.where(ok, hi, mid)
        cont = (jnp.max(hi - lo).astype(jnp.int32) > 1).astype(jnp.int32)
        return lo, hi, cont

    tb, _, _ = jax.lax.while_loop(_cond, _step, (lo0, hi0, jnp.int32(1)))
    tb_f = jax.lax.bitcast_convert_type(tb, jnp.float32)

    gt = e_all > tb_f
    tie = e_all == tb_f
    c_gt = jnp.sum(gt.astype(jnp.float32), axis=1, keepdims=True)
    k_rem = kf - c_gt
    prefix = _exclusive_prefix(tie.astype(jnp.bfloat16), srows, cols)
    sel_mask = gt | (tie & (prefix < k_rem))

    sel = jnp.where(sel_mask, e_all, 0.0)
    s = jnp.sum(sel, axis=1, keepdims=True)
    scale = fw_all / (s + 1e-8 * z_all)
    weighted = sel * scale
    w4 = weighted.reshape(_H, rows, cols)
    out_ref[...] = w4[0] + w4[1] + w4[2] + w4[3]


def _fused_topk_adj(e1, e2, fw, invt, *, topk, block_rows):
    h, n, d = e1.shape
    grid = (n // block_rows,)
    return pl.pallas_call(
        functools.partial(_block_body, topk=topk),
        grid=grid,
        in_specs=[
            pl.BlockSpec(memory_space=pltpu.SMEM),
            pl.BlockSpec(memory_space=pltpu.SMEM),
            pl.BlockSpec((h, block_rows, d), lambda i: (0, i, 0)),
            pl.BlockSpec((h, d, n), lambda i: (0, 0, 0)),
        ],
        out_specs=pl.BlockSpec((block_rows, n), lambda i: (i, 0)),
        out_shape=jax.ShapeDtypeStruct((n, n), jnp.float32),
        scratch_shapes=[pltpu.VMEM((h * block_rows, n), jnp.float32)],
    )(fw, invt, e1, e2)


def kernel(node_embeddings1, node_embeddings2, temperature, fusion_weights):
    temp = jnp.clip(temperature, 0.1, 2.0)
    invt = 1.0 / temp
    fw = jax.nn.softmax(fusion_weights, axis=0)
    return _fused_topk_adj(
        node_embeddings1,
        node_embeddings2,
        fw,
        invt,
        topk=32,
        block_rows=256,
    )
